# baseline (speedup 1.0000x reference)
import functools

import jax
import jax.numpy as jnp
from jax import lax
from jax.experimental import pallas as pl
from jax.experimental.pallas import tpu as pltpu
from jax.experimental.pallas import tpu_sc as plsc

N = 10000
E = 320000
T = 100000
H = 128
NT = 7

NB = 10
BN = N // NB

CHUNK = 64
PCHUNK = 128
N_WORKERS = 32
E_PAD = 327680
T_PAD = 102400
N_PAD = 10240
ROWS_PER_TILE = N_PAD // 16

NCH_E = E_PAD // N_WORKERS // CHUNK
NCH_P = T_PAD // N_WORKERS // PCHUNK

_sc_mesh = plsc.VectorSubcoreMesh(core_axis_name="c", subcore_axis_name="s")


def _tf_first_body(h_ref, wm_ref, wgd_ref, wgs_ref, ws_ref, bs_ref,
                   ft_ref, gt_ref, skip_ref):
    h = h_ref[...]
    ft_ref[:, 0:H] = jnp.dot(h, wm_ref[0], preferred_element_type=jnp.float32)
    ft_ref[:, H:2 * H] = jnp.dot(h, wgs_ref[0],
                                 preferred_element_type=jnp.float32)
    gt_ref[...] = jnp.dot(h, wgd_ref[0], preferred_element_type=jnp.float32)

    @pl.when(pl.program_id(1) == 0)
    def _():
        skip_ref[...] = (jnp.dot(h, ws_ref[...], preferred_element_type=jnp.float32)
                         + bs_ref[...])


_W_SPECS = [
    pl.BlockSpec((1, H, H), lambda nb, t: (t, 0, 0)),
    pl.BlockSpec((1, H, H), lambda nb, t: (t, 0, 0)),
    pl.BlockSpec((1, H, H), lambda nb, t: (t, 0, 0)),
    pl.BlockSpec((H, H), lambda nb, t: (0, 0)),
    pl.BlockSpec((1, H), lambda nb, t: (0, 0)),
]
_TAB_SPECS = [
    pl.BlockSpec((BN, 2 * H), lambda nb, t: (t * NB + nb, 0)),
    pl.BlockSpec((BN, H), lambda nb, t: (t * NB + nb, 0)),
]
_TAB_SHAPES = [
    jax.ShapeDtypeStruct((NT * N, 2 * H), jnp.float32),
    jax.ShapeDtypeStruct((NT * N, H), jnp.float32),
]
_NODE_SPEC = pl.BlockSpec((BN, H), lambda nb, t: (nb, 0))


def _transform_first(x, lp):
    return pl.pallas_call(
        _tf_first_body,
        grid=(NB, NT),
        in_specs=[_NODE_SPEC] + _W_SPECS,
        out_specs=_TAB_SPECS + [_NODE_SPEC],
        out_shape=_TAB_SHAPES + [jax.ShapeDtypeStruct((N, H), jnp.float32)],
    )(x, lp['W_msg'], lp['W_gd'], lp['W_gs'], lp['W_skip'],
      lp['b_skip'].reshape(1, H))


def _tf_next_body(skip_ref, agg_ref, lng_ref, lnb_ref,
                  wm_ref, wgd_ref, wgs_ref, ws_ref, bs_ref,
                  ft_ref, gt_ref, skipo_ref, hsave_ref, h_scr):
    @pl.when(pl.program_id(1) == 0)
    def _():
        u = jnp.maximum(skip_ref[...] + agg_ref[0] + agg_ref[1], 0.0)
        m = jnp.mean(u, axis=-1, keepdims=True)
        v = jnp.mean((u - m) ** 2, axis=-1, keepdims=True)
        hh = (u - m) * lax.rsqrt(v + 1e-5) * lng_ref[...] + lnb_ref[...]
        h_scr[...] = hh
        hsave_ref[...] = hh
        skipo_ref[...] = (jnp.dot(hh, ws_ref[...], preferred_element_type=jnp.float32)
                          + bs_ref[...])

    h = h_scr[...]
    ft_ref[:, 0:H] = jnp.dot(h, wm_ref[0], preferred_element_type=jnp.float32)
    ft_ref[:, H:2 * H] = jnp.dot(h, wgs_ref[0],
                                 preferred_element_type=jnp.float32)
    gt_ref[...] = jnp.dot(h, wgd_ref[0], preferred_element_type=jnp.float32)


_AGG_SPEC = pl.BlockSpec((2, BN, H), lambda nb, *_: (0, nb, 0))


def _transform_next(skip_prev, agg, ln_g, ln_b, lp):
    nh = jax.ShapeDtypeStruct((N, H), jnp.float32)
    return pl.pallas_call(
        _tf_next_body,
        grid=(NB, NT),
        in_specs=[
            _NODE_SPEC, _AGG_SPEC,
            pl.BlockSpec((1, H), lambda nb, t: (0, 0)),
            pl.BlockSpec((1, H), lambda nb, t: (0, 0)),
        ] + _W_SPECS,
        out_specs=_TAB_SPECS + [_NODE_SPEC, _NODE_SPEC],
        out_shape=_TAB_SHAPES + [nh, nh],
        scratch_shapes=[pltpu.VMEM((BN, H), jnp.float32)],
    )(skip_prev, agg, ln_g.reshape(1, H), ln_b.reshape(1, H),
      lp['W_msg'], lp['W_gd'], lp['W_gs'], lp['W_skip'],
      lp['b_skip'].reshape(1, H))


def _jk_body(skip_ref, agg_ref, h1_ref, h2_ref, jkw_ref, jkb_ref,
             p1a_ref, p1b_ref, a_ref, b_ref):
    u = skip_ref[...] + agg_ref[0] + agg_ref[1]
    hf = (jnp.dot(h1_ref[...], jkw_ref[0], preferred_element_type=jnp.float32)
          + jnp.dot(h2_ref[...], jkw_ref[1], preferred_element_type=jnp.float32)
          + jnp.dot(u, jkw_ref[2], preferred_element_type=jnp.float32)
          + jkb_ref[...])
    a_ref[...] = jnp.dot(hf, p1a_ref[...], preferred_element_type=jnp.float32)
    b_ref[...] = jnp.dot(hf, p1b_ref[...], preferred_element_type=jnp.float32)


def _jk_project(skip2, agg, h1, h2, jk_W, jk_b, p1a, p1b):
    nh = jax.ShapeDtypeStruct((N, H), jnp.float32)
    blk = pl.BlockSpec((BN, H), lambda nb: (nb, 0))
    return pl.pallas_call(
        _jk_body,
        grid=(NB,),
        in_specs=[
            blk, _AGG_SPEC, blk, blk,
            pl.BlockSpec((3, H, H), lambda nb: (0, 0, 0)),
            pl.BlockSpec((1, H), lambda nb: (0, 0)),
            pl.BlockSpec((H, H), lambda nb: (0, 0)),
            pl.BlockSpec((H, H), lambda nb: (0, 0)),
        ],
        out_specs=[blk, blk],
        out_shape=[nh, nh],
    )(skip2, agg, h1, h2, jk_W.reshape(3, H, H), jk_b.reshape(1, H), p1a, p1b)


def _final_body(g_ref, pit_ref, ons_ref, wp_ref, wo_ref, b1_ref,
                w2_ref, b2_ref, w3_ref, b3_ref, out_ref):
    c = pit_ref[...] * wp_ref[...]
    c = c + ons_ref[:, 0:1] * wo_ref[0:1, :] + ons_ref[:, 1:2] * wo_ref[1:2, :]
    z1 = jnp.maximum(g_ref[...] + c + b1_ref[...], 0.0)
    z2 = jnp.maximum(jnp.dot(z1, w2_ref[...], preferred_element_type=jnp.float32)
                     + b2_ref[...], 0.0)
    o = jnp.sum(z2 * w3_ref[...], axis=1, keepdims=True) + b3_ref[...]
    out_ref[...] = 1.0 / (1.0 + jnp.exp(-o))


def _final_mlp(g, pitch, onset, wp, wo, b1, w2, b2, w3, b3):
    return pl.pallas_call(
        _final_body,
        grid=(T // BN,),
        in_specs=[
            pl.BlockSpec((BN, H), lambda i: (i, 0)),
            pl.BlockSpec((BN, 1), lambda i: (i, 0)),
            pl.BlockSpec((BN, 2), lambda i: (i, 0)),
            pl.BlockSpec((1, H), lambda i: (0, 0)),
            pl.BlockSpec((2, H), lambda i: (0, 0)),
            pl.BlockSpec((1, H), lambda i: (0, 0)),
            pl.BlockSpec((H, H // 2), lambda i: (0, 0)),
            pl.BlockSpec((1, H // 2), lambda i: (0, 0)),
            pl.BlockSpec((1, H // 2), lambda i: (0, 0)),
            pl.BlockSpec((1, 1), lambda i: (0, 0)),
        ],
        out_specs=pl.BlockSpec((BN, 1), lambda i: (i, 0)),
        out_shape=jax.ShapeDtypeStruct((T, 1), jnp.float32),
    )(g, pitch, onset, wp, wo, b1, w2, b2, w3, b3)


@functools.partial(
    pl.kernel,
    out_type=jax.ShapeDtypeStruct((2, N_PAD, H), jnp.float32),
    mesh=_sc_mesh,
    scratch_types=[
        pltpu.VMEM((3 * CHUNK,), jnp.int32),
        pltpu.VMEM((CHUNK, 2 * H), jnp.float32),
        pltpu.VMEM((CHUNK, H), jnp.float32),
        pltpu.VMEM_SHARED((N_PAD, H), jnp.float32),
        pltpu.SemaphoreType.DMA,
    ],
)
def _edge_kernel(ft_hbm, gt_hbm, idx_hbm, agg_hbm,
                 ib, fbuf, gbuf, acc_sh, sem):
    cid = lax.axis_index("c")
    sid = lax.axis_index("s")
    wid = sid * 2 + cid
    row0 = sid * ROWS_PER_TILE

    zero16 = jnp.zeros((16,), jnp.float32)

    def _zrow(r, carry):
        for v in range(H // 16):
            gbuf[r, pl.ds(v * 16, 16)] = zero16
        return carry

    lax.fori_loop(0, CHUNK, _zrow, 0)
    for k in range(ROWS_PER_TILE // CHUNK):
        pltpu.sync_copy(gbuf, acc_sh.at[pl.ds(row0 + k * CHUNK, CHUNK)])
    plsc.subcore_barrier()

    def _chunk(i, carry):
        pltpu.sync_copy(
            idx_hbm.at[pl.ds((wid * NCH_E + i) * (3 * CHUNK), 3 * CHUNK)], ib)
        cf = pltpu.async_copy(ft_hbm.at[ib.at[pl.ds(0, CHUNK)]], fbuf, sem)
        cg = pltpu.async_copy(gt_hbm.at[ib.at[pl.ds(CHUNK, CHUNK)]], gbuf, sem)
        cf.wait()
        cg.wait()

        def _row(r, cy):
            for v in range(H // 16):
                sl = pl.ds(v * 16, 16)
                slg = pl.ds(H + v * 16, 16)
                pre = gbuf[r, sl] + fbuf[r, slg]
                gate = 1.0 / (1.0 + jnp.exp(-pre))
                gbuf[r, sl] = gate * fbuf[r, sl]
            return cy

        lax.fori_loop(0, CHUNK, _row, 0)
        pltpu.sync_copy(gbuf, acc_sh.at[ib.at[pl.ds(2 * CHUNK, CHUNK)]],
                        add=True)
        return carry

    lax.fori_loop(0, NCH_E, _chunk, 0)
    plsc.subcore_barrier()
    pltpu.sync_copy(acc_sh.at[pl.ds(row0, ROWS_PER_TILE)],
                    agg_hbm.at[cid, pl.ds(row0, ROWS_PER_TILE)])


@functools.partial(
    pl.kernel,
    out_type=jax.ShapeDtypeStruct((T_PAD, H), jnp.float32),
    mesh=_sc_mesh,
    scratch_types=[
        pltpu.VMEM((NCH_P * PCHUNK,), jnp.int32),
        pltpu.VMEM((NCH_P * PCHUNK,), jnp.int32),
        pltpu.VMEM((PCHUNK, H), jnp.float32),
        pltpu.VMEM((PCHUNK, H), jnp.float32),
        pltpu.VMEM((PCHUNK, H), jnp.float32),
        pltpu.VMEM((PCHUNK, H), jnp.float32),
        pltpu.SemaphoreType.DMA,
        pltpu.SemaphoreType.DMA,
        pltpu.SemaphoreType.DMA,
        pltpu.SemaphoreType.DMA,
    ],
)
def _pred_gather_kernel(a_hbm, b_hbm, si_hbm, di_hbm, gout_hbm,
                        siloc, diloc, ga0, ga1, gb0, gb1,
                        sa0, sa1, sb0, sb1):
    cid = lax.axis_index("c")
    sid = lax.axis_index("s")
    wid = sid * 2 + cid
    ga = (ga0, ga1)
    gb = (gb0, gb1)
    sa = (sa0, sa1)
    sb = (sb0, sb1)

    npt = NCH_P * PCHUNK
    pltpu.sync_copy(si_hbm.at[pl.ds(wid * npt, npt)], siloc)
    pltpu.sync_copy(di_hbm.at[pl.ds(wid * npt, npt)], diloc)

    descs = {}

    def _issue(i):
        b = i % 2
        descs[(i, 'a')] = pltpu.async_copy(
            a_hbm.at[siloc.at[pl.ds(i * PCHUNK, PCHUNK)]], ga[b], sa[b])
        descs[(i, 'b')] = pltpu.async_copy(
            b_hbm.at[diloc.at[pl.ds(i * PCHUNK, PCHUNK)]], gb[b], sb[b])

    _issue(0)
    base = wid * NCH_P * PCHUNK
    for i in range(NCH_P):
        b = i % 2
        if i < NCH_P - 1:
            _issue(i + 1)
        descs[(i, 'a')].wait()
        descs[(i, 'b')].wait()

        def _row(r, cy):
            for v in range(H // 16):
                sl = pl.ds(v * 16, 16)
                ga[b][r, sl] = ga[b][r, sl] + gb[b][r, sl]
            return cy

        lax.fori_loop(0, PCHUNK, _row, 0)
        pltpu.sync_copy(ga[b], gout_hbm.at[pl.ds(base + i * PCHUNK, PCHUNK)])


def kernel(target_edge_index, x, embed_edge_index, edge_type, pitch_score,
           onset_score, params):
    src = embed_edge_index[0].astype(jnp.int32)
    dst = embed_edge_index[1].astype(jnp.int32)
    et = edge_type.astype(jnp.int32)

    isrc = et * N + src
    idst = et * N + dst

    epad = E_PAD - E
    zpad = jnp.zeros((epad,), jnp.int32)
    isrc_p = jnp.concatenate([isrc, zpad]).reshape(-1, CHUNK)
    idst_p = jnp.concatenate([idst, zpad]).reshape(-1, CHUNK)
    dnode_p = jnp.concatenate(
        [dst, jnp.full((epad,), N, jnp.int32)]).reshape(-1, CHUNK)
    idx_p = jnp.stack([isrc_p, idst_p, dnode_p], axis=1).reshape(-1)

    tpad = T_PAD - T
    tz = jnp.zeros((tpad,), jnp.int32)
    si_p = jnp.concatenate([target_edge_index[0].astype(jnp.int32), tz])
    di_p = jnp.concatenate([target_edge_index[1].astype(jnp.int32), tz])

    layers = params['layers']
    ln_g, ln_b = params['ln_g'], params['ln_b']

    ft, gt, skip = _transform_first(x, layers[0])
    agg = _edge_kernel(ft, gt, idx_p)

    ft, gt, skip, h1 = _transform_next(skip, agg, ln_g, ln_b, layers[1])
    agg = _edge_kernel(ft, gt, idx_p)

    ft, gt, skip, h2 = _transform_next(skip, agg, ln_g, ln_b, layers[2])
    agg = _edge_kernel(ft, gt, idx_p)

    p1_W = params['p1_W']
    a_tab, b_tab = _jk_project(skip, agg, h1, h2, params['jk_W'],
                               params['jk_b'], p1_W[:H], p1_W[H:2 * H])

    g = _pred_gather_kernel(a_tab, b_tab, si_p, di_p)

    return _final_mlp(
        g, pitch_score, onset_score,
        p1_W[2 * H:2 * H + 1], p1_W[2 * H + 1:2 * H + 3],
        params['p1_b'].reshape(1, H),
        params['p2_W'], params['p2_b'].reshape(1, H // 2),
        params['p3_W'].reshape(1, H // 2), params['p3_b'].reshape(1, 1))


# speedup vs baseline: 1.5226x; 1.5226x over previous
import functools

import jax
import jax.numpy as jnp
from jax import lax
from jax.experimental import pallas as pl
from jax.experimental.pallas import tpu as pltpu
from jax.experimental.pallas import tpu_sc as plsc

N = 10000
E = 320000
T = 100000
H = 128
NT = 7

NB = 10
BN = N // NB

CHUNK = 64
PCHUNK = 128
N_WORKERS = 32
E_PAD = 327680
T_PAD = 102400
N_PAD = 10240
ROWS_PER_TILE = N_PAD // 16

NCH_E = E_PAD // N_WORKERS // CHUNK
NCH_P = T_PAD // N_WORKERS // PCHUNK

_sc_mesh = plsc.VectorSubcoreMesh(core_axis_name="c", subcore_axis_name="s")


def _tf_first_body(h_ref, wm_ref, wgd_ref, wgs_ref, ws_ref, bs_ref,
                   tm_ref, tgd_ref, tgs_ref, skip_ref):
    h = h_ref[...]
    tm_ref[...] = jnp.dot(h, wm_ref[0], preferred_element_type=jnp.float32)
    tgd_ref[...] = jnp.dot(h, wgd_ref[0], preferred_element_type=jnp.float32)
    tgs_ref[...] = jnp.dot(h, wgs_ref[0], preferred_element_type=jnp.float32)

    @pl.when(pl.program_id(1) == 0)
    def _():
        skip_ref[...] = (jnp.dot(h, ws_ref[...], preferred_element_type=jnp.float32)
                         + bs_ref[...])


_W_SPECS = [
    pl.BlockSpec((1, H, H), lambda nb, t: (t, 0, 0)),
    pl.BlockSpec((1, H, H), lambda nb, t: (t, 0, 0)),
    pl.BlockSpec((1, H, H), lambda nb, t: (t, 0, 0)),
    pl.BlockSpec((H, H), lambda nb, t: (0, 0)),
    pl.BlockSpec((1, H), lambda nb, t: (0, 0)),
]
_TAB_SPECS = [
    pl.BlockSpec((BN, H), lambda nb, t: (t * NB + nb, 0)),
    pl.BlockSpec((BN, H), lambda nb, t: (t * NB + nb, 0)),
    pl.BlockSpec((BN, H), lambda nb, t: (t * NB + nb, 0)),
]
_TAB_SHAPES = [
    jax.ShapeDtypeStruct((NT * N, H), jnp.float32),
    jax.ShapeDtypeStruct((NT * N, H), jnp.float32),
    jax.ShapeDtypeStruct((NT * N, H), jnp.float32),
]
_NODE_SPEC = pl.BlockSpec((BN, H), lambda nb, t: (nb, 0))


def _transform_first(x, lp):
    return pl.pallas_call(
        _tf_first_body,
        grid=(NB, NT),
        in_specs=[_NODE_SPEC] + _W_SPECS,
        out_specs=_TAB_SPECS + [_NODE_SPEC],
        out_shape=_TAB_SHAPES + [jax.ShapeDtypeStruct((N, H), jnp.float32)],
    )(x, lp['W_msg'], lp['W_gd'], lp['W_gs'], lp['W_skip'],
      lp['b_skip'].reshape(1, H))


def _tf_next_body(skip_ref, agg_ref, lng_ref, lnb_ref,
                  wm_ref, wgd_ref, wgs_ref, ws_ref, bs_ref,
                  tm_ref, tgd_ref, tgs_ref, skipo_ref, hsave_ref, h_scr):
    @pl.when(pl.program_id(1) == 0)
    def _():
        u = jnp.maximum(skip_ref[...] + agg_ref[0] + agg_ref[1], 0.0)
        m = jnp.mean(u, axis=-1, keepdims=True)
        v = jnp.mean((u - m) ** 2, axis=-1, keepdims=True)
        hh = (u - m) * lax.rsqrt(v + 1e-5) * lng_ref[...] + lnb_ref[...]
        h_scr[...] = hh
        hsave_ref[...] = hh
        skipo_ref[...] = (jnp.dot(hh, ws_ref[...], preferred_element_type=jnp.float32)
                          + bs_ref[...])

    h = h_scr[...]
    tm_ref[...] = jnp.dot(h, wm_ref[0], preferred_element_type=jnp.float32)
    tgd_ref[...] = jnp.dot(h, wgd_ref[0], preferred_element_type=jnp.float32)
    tgs_ref[...] = jnp.dot(h, wgs_ref[0], preferred_element_type=jnp.float32)


_AGG_SPEC = pl.BlockSpec((2, BN, H), lambda nb, *_: (0, nb, 0))


def _transform_next(skip_prev, agg, ln_g, ln_b, lp):
    nh = jax.ShapeDtypeStruct((N, H), jnp.float32)
    return pl.pallas_call(
        _tf_next_body,
        grid=(NB, NT),
        in_specs=[
            _NODE_SPEC, _AGG_SPEC,
            pl.BlockSpec((1, H), lambda nb, t: (0, 0)),
            pl.BlockSpec((1, H), lambda nb, t: (0, 0)),
        ] + _W_SPECS,
        out_specs=_TAB_SPECS + [_NODE_SPEC, _NODE_SPEC],
        out_shape=_TAB_SHAPES + [nh, nh],
        scratch_shapes=[pltpu.VMEM((BN, H), jnp.float32)],
    )(skip_prev, agg, ln_g.reshape(1, H), ln_b.reshape(1, H),
      lp['W_msg'], lp['W_gd'], lp['W_gs'], lp['W_skip'],
      lp['b_skip'].reshape(1, H))


def _jk_body(skip_ref, agg_ref, h1_ref, h2_ref, jkw_ref, jkb_ref,
             p1a_ref, p1b_ref, a_ref, b_ref):
    u = skip_ref[...] + agg_ref[0] + agg_ref[1]
    hf = (jnp.dot(h1_ref[...], jkw_ref[0], preferred_element_type=jnp.float32)
          + jnp.dot(h2_ref[...], jkw_ref[1], preferred_element_type=jnp.float32)
          + jnp.dot(u, jkw_ref[2], preferred_element_type=jnp.float32)
          + jkb_ref[...])
    a_ref[...] = jnp.dot(hf, p1a_ref[...], preferred_element_type=jnp.float32)
    b_ref[...] = jnp.dot(hf, p1b_ref[...], preferred_element_type=jnp.float32)


def _jk_project(skip2, agg, h1, h2, jk_W, jk_b, p1a, p1b):
    nh = jax.ShapeDtypeStruct((N, H), jnp.float32)
    blk = pl.BlockSpec((BN, H), lambda nb: (nb, 0))
    return pl.pallas_call(
        _jk_body,
        grid=(NB,),
        in_specs=[
            blk, _AGG_SPEC, blk, blk,
            pl.BlockSpec((3, H, H), lambda nb: (0, 0, 0)),
            pl.BlockSpec((1, H), lambda nb: (0, 0)),
            pl.BlockSpec((H, H), lambda nb: (0, 0)),
            pl.BlockSpec((H, H), lambda nb: (0, 0)),
        ],
        out_specs=[blk, blk],
        out_shape=[nh, nh],
    )(skip2, agg, h1, h2, jk_W.reshape(3, H, H), jk_b.reshape(1, H), p1a, p1b)


def _final_body(g_ref, pit_ref, ons_ref, wp_ref, wo_ref, b1_ref,
                w2_ref, b2_ref, w3_ref, b3_ref, out_ref):
    c = pit_ref[...] * wp_ref[...]
    c = c + ons_ref[:, 0:1] * wo_ref[0:1, :] + ons_ref[:, 1:2] * wo_ref[1:2, :]
    z1 = jnp.maximum(g_ref[...] + c + b1_ref[...], 0.0)
    z2 = jnp.maximum(jnp.dot(z1, w2_ref[...], preferred_element_type=jnp.float32)
                     + b2_ref[...], 0.0)
    o = jnp.sum(z2 * w3_ref[...], axis=1, keepdims=True) + b3_ref[...]
    out_ref[...] = 1.0 / (1.0 + jnp.exp(-o))


def _final_mlp(g, pitch, onset, wp, wo, b1, w2, b2, w3, b3):
    return pl.pallas_call(
        _final_body,
        grid=(T // BN,),
        in_specs=[
            pl.BlockSpec((BN, H), lambda i: (i, 0)),
            pl.BlockSpec((BN, 1), lambda i: (i, 0)),
            pl.BlockSpec((BN, 2), lambda i: (i, 0)),
            pl.BlockSpec((1, H), lambda i: (0, 0)),
            pl.BlockSpec((2, H), lambda i: (0, 0)),
            pl.BlockSpec((1, H), lambda i: (0, 0)),
            pl.BlockSpec((H, H // 2), lambda i: (0, 0)),
            pl.BlockSpec((1, H // 2), lambda i: (0, 0)),
            pl.BlockSpec((1, H // 2), lambda i: (0, 0)),
            pl.BlockSpec((1, 1), lambda i: (0, 0)),
        ],
        out_specs=pl.BlockSpec((BN, 1), lambda i: (i, 0)),
        out_shape=jax.ShapeDtypeStruct((T, 1), jnp.float32),
    )(g, pitch, onset, wp, wo, b1, w2, b2, w3, b3)


@functools.partial(
    pl.kernel,
    out_type=jax.ShapeDtypeStruct((2, N_PAD, H), jnp.float32),
    mesh=_sc_mesh,
    scratch_types=[
        pltpu.VMEM((3 * CHUNK,), jnp.int32),
        pltpu.VMEM((CHUNK, H), jnp.float32),
        pltpu.VMEM((CHUNK, H), jnp.float32),
        pltpu.VMEM((CHUNK, H), jnp.float32),
        pltpu.VMEM((CHUNK, H), jnp.float32),
        pltpu.VMEM_SHARED((N_PAD, H), jnp.float32),
        pltpu.SemaphoreType.DMA,
        pltpu.SemaphoreType.DMA,
        pltpu.SemaphoreType.DMA,
    ],
)
def _edge_kernel(tm_hbm, tgd_hbm, tgs_hbm, idx_hbm, agg_hbm,
                 ib, rm_v, rgd_v, rgs_v, msg_v, acc_sh, sem_m, sem_gd, sem_gs):
    cid = lax.axis_index("c")
    sid = lax.axis_index("s")
    wid = sid * 2 + cid
    row0 = sid * ROWS_PER_TILE

    zero16 = jnp.zeros((16,), jnp.float32)

    def _zrow(r, carry):
        for v in range(H // 16):
            msg_v[r, pl.ds(v * 16, 16)] = zero16
        return carry

    lax.fori_loop(0, CHUNK, _zrow, 0)
    for k in range(ROWS_PER_TILE // CHUNK):
        pltpu.sync_copy(msg_v, acc_sh.at[pl.ds(row0 + k * CHUNK, CHUNK)])
    plsc.subcore_barrier()

    def _chunk(i, carry):
        pltpu.sync_copy(
            idx_hbm.at[pl.ds((wid * NCH_E + i) * (3 * CHUNK), 3 * CHUNK)], ib)
        isl = ib.at[pl.ds(0, CHUNK)]
        cm = pltpu.async_copy(tm_hbm.at[isl], rm_v, sem_m)
        cgd = pltpu.async_copy(tgd_hbm.at[ib.at[pl.ds(CHUNK, CHUNK)]],
                               rgd_v, sem_gd)
        cgs = pltpu.async_copy(tgs_hbm.at[isl], rgs_v, sem_gs)
        cm.wait()
        cgd.wait()
        cgs.wait()

        def _row(r, cy):
            for v in range(H // 16):
                sl = pl.ds(v * 16, 16)
                pre = rgd_v[r, sl] + rgs_v[r, sl]
                gate = 1.0 / (1.0 + jnp.exp(-pre))
                msg_v[r, sl] = gate * rm_v[r, sl]
            return cy

        lax.fori_loop(0, CHUNK, _row, 0)
        pltpu.sync_copy(msg_v, acc_sh.at[ib.at[pl.ds(2 * CHUNK, CHUNK)]],
                        add=True)
        return carry

    lax.fori_loop(0, NCH_E, _chunk, 0)
    plsc.subcore_barrier()
    pltpu.sync_copy(acc_sh.at[pl.ds(row0, ROWS_PER_TILE)],
                    agg_hbm.at[cid, pl.ds(row0, ROWS_PER_TILE)])


@functools.partial(
    pl.kernel,
    out_type=jax.ShapeDtypeStruct((T_PAD, H), jnp.float32),
    mesh=_sc_mesh,
    scratch_types=[
        pltpu.VMEM((NCH_P * PCHUNK,), jnp.int32),
        pltpu.VMEM((NCH_P * PCHUNK,), jnp.int32),
        pltpu.VMEM((PCHUNK, H), jnp.float32),
        pltpu.VMEM((PCHUNK, H), jnp.float32),
        pltpu.VMEM((PCHUNK, H), jnp.float32),
        pltpu.VMEM((PCHUNK, H), jnp.float32),
        pltpu.SemaphoreType.DMA,
        pltpu.SemaphoreType.DMA,
        pltpu.SemaphoreType.DMA,
        pltpu.SemaphoreType.DMA,
    ],
)
def _pred_gather_kernel(a_hbm, b_hbm, si_hbm, di_hbm, gout_hbm,
                        siloc, diloc, ga0, ga1, gb0, gb1,
                        sa0, sa1, sb0, sb1):
    cid = lax.axis_index("c")
    sid = lax.axis_index("s")
    wid = sid * 2 + cid
    ga = (ga0, ga1)
    gb = (gb0, gb1)
    sa = (sa0, sa1)
    sb = (sb0, sb1)

    npt = NCH_P * PCHUNK
    pltpu.sync_copy(si_hbm.at[pl.ds(wid * npt, npt)], siloc)
    pltpu.sync_copy(di_hbm.at[pl.ds(wid * npt, npt)], diloc)

    descs = {}

    def _issue(i):
        b = i % 2
        descs[(i, 'a')] = pltpu.async_copy(
            a_hbm.at[siloc.at[pl.ds(i * PCHUNK, PCHUNK)]], ga[b], sa[b])
        descs[(i, 'b')] = pltpu.async_copy(
            b_hbm.at[diloc.at[pl.ds(i * PCHUNK, PCHUNK)]], gb[b], sb[b])

    _issue(0)
    base = wid * NCH_P * PCHUNK
    for i in range(NCH_P):
        b = i % 2
        if i < NCH_P - 1:
            _issue(i + 1)
        descs[(i, 'a')].wait()
        descs[(i, 'b')].wait()

        def _row(r, cy):
            for v in range(H // 16):
                sl = pl.ds(v * 16, 16)
                ga[b][r, sl] = ga[b][r, sl] + gb[b][r, sl]
            return cy

        lax.fori_loop(0, PCHUNK, _row, 0)
        pltpu.sync_copy(ga[b], gout_hbm.at[pl.ds(base + i * PCHUNK, PCHUNK)])


def kernel(target_edge_index, x, embed_edge_index, edge_type, pitch_score,
           onset_score, params):
    src = embed_edge_index[0].astype(jnp.int32)
    dst = embed_edge_index[1].astype(jnp.int32)
    et = edge_type.astype(jnp.int32)

    isrc = et * N + src
    idst = et * N + dst

    epad = E_PAD - E
    zpad = jnp.zeros((epad,), jnp.int32)
    isrc_p = jnp.concatenate([isrc, zpad]).reshape(-1, CHUNK)
    idst_p = jnp.concatenate([idst, zpad]).reshape(-1, CHUNK)
    dnode_p = jnp.concatenate(
        [dst, jnp.full((epad,), N, jnp.int32)]).reshape(-1, CHUNK)
    idx_p = jnp.stack([isrc_p, idst_p, dnode_p], axis=1).reshape(-1)

    tpad = T_PAD - T
    tz = jnp.zeros((tpad,), jnp.int32)
    si_p = jnp.concatenate([target_edge_index[0].astype(jnp.int32), tz])
    di_p = jnp.concatenate([target_edge_index[1].astype(jnp.int32), tz])

    layers = params['layers']
    ln_g, ln_b = params['ln_g'], params['ln_b']

    tm, tgd, tgs, skip = _transform_first(x, layers[0])
    agg = _edge_kernel(tm, tgd, tgs, idx_p)

    tm, tgd, tgs, skip, h1 = _transform_next(skip, agg, ln_g, ln_b, layers[1])
    agg = _edge_kernel(tm, tgd, tgs, idx_p)

    tm, tgd, tgs, skip, h2 = _transform_next(skip, agg, ln_g, ln_b, layers[2])
    agg = _edge_kernel(tm, tgd, tgs, idx_p)

    p1_W = params['p1_W']
    a_tab, b_tab = _jk_project(skip, agg, h1, h2, params['jk_W'],
                               params['jk_b'], p1_W[:H], p1_W[H:2 * H])

    g = _pred_gather_kernel(a_tab, b_tab, si_p, di_p)

    return _final_mlp(
        g, pitch_score, onset_score,
        p1_W[2 * H:2 * H + 1], p1_W[2 * H + 1:2 * H + 3],
        params['p1_b'].reshape(1, H),
        params['p2_W'], params['p2_b'].reshape(1, H // 2),
        params['p3_W'].reshape(1, H // 2), params['p3_b'].reshape(1, 1))


# speedup vs baseline: 2.2175x; 1.4565x over previous
import functools

import jax
import jax.numpy as jnp
from jax import lax
from jax.experimental import pallas as pl
from jax.experimental.pallas import tpu as pltpu
from jax.experimental.pallas import tpu_sc as plsc

N = 10000
E = 320000
T = 100000
H = 128
NT = 7

NB = 10
BN = N // NB

CHUNK = 64
PCHUNK = 128
N_WORKERS = 32
E_PAD = 327680
T_PAD = 102400
N_PAD = 10240
ROWS_PER_TILE = N_PAD // 16

NCH_E = E_PAD // N_WORKERS // CHUNK
NCH_P = T_PAD // N_WORKERS // PCHUNK

_sc_mesh = plsc.VectorSubcoreMesh(core_axis_name="c", subcore_axis_name="s")


def _tf_first_body(h_ref, wm_ref, wgd_ref, wgs_ref, ws_ref, bs_ref,
                   tm_ref, tgd_ref, tgs_ref, skip_ref):
    h = h_ref[...]
    tm_ref[...] = jnp.dot(h, wm_ref[0], preferred_element_type=jnp.float32)
    tgd_ref[...] = jnp.dot(h, wgd_ref[0], preferred_element_type=jnp.float32)
    tgs_ref[...] = jnp.dot(h, wgs_ref[0], preferred_element_type=jnp.float32)

    @pl.when(pl.program_id(1) == 0)
    def _():
        skip_ref[...] = (jnp.dot(h, ws_ref[...], preferred_element_type=jnp.float32)
                         + bs_ref[...])


_W_SPECS = [
    pl.BlockSpec((1, H, H), lambda nb, t: (t, 0, 0)),
    pl.BlockSpec((1, H, H), lambda nb, t: (t, 0, 0)),
    pl.BlockSpec((1, H, H), lambda nb, t: (t, 0, 0)),
    pl.BlockSpec((H, H), lambda nb, t: (0, 0)),
    pl.BlockSpec((1, H), lambda nb, t: (0, 0)),
]
_TAB_SPECS = [
    pl.BlockSpec((BN, H), lambda nb, t: (t * NB + nb, 0)),
    pl.BlockSpec((BN, H), lambda nb, t: (t * NB + nb, 0)),
    pl.BlockSpec((BN, H), lambda nb, t: (t * NB + nb, 0)),
]
_TAB_SHAPES = [
    jax.ShapeDtypeStruct((NT * N, H), jnp.float32),
    jax.ShapeDtypeStruct((NT * N, H), jnp.float32),
    jax.ShapeDtypeStruct((NT * N, H), jnp.float32),
]
_NODE_SPEC = pl.BlockSpec((BN, H), lambda nb, t: (nb, 0))


def _transform_first(x, lp):
    return pl.pallas_call(
        _tf_first_body,
        grid=(NB, NT),
        in_specs=[_NODE_SPEC] + _W_SPECS,
        out_specs=_TAB_SPECS + [_NODE_SPEC],
        out_shape=_TAB_SHAPES + [jax.ShapeDtypeStruct((N, H), jnp.float32)],
    )(x, lp['W_msg'], lp['W_gd'], lp['W_gs'], lp['W_skip'],
      lp['b_skip'].reshape(1, H))


def _tf_next_body(skip_ref, agg_ref, lng_ref, lnb_ref,
                  wm_ref, wgd_ref, wgs_ref, ws_ref, bs_ref,
                  tm_ref, tgd_ref, tgs_ref, skipo_ref, hsave_ref, h_scr):
    @pl.when(pl.program_id(1) == 0)
    def _():
        u = jnp.maximum(skip_ref[...] + agg_ref[0] + agg_ref[1], 0.0)
        m = jnp.mean(u, axis=-1, keepdims=True)
        v = jnp.mean((u - m) ** 2, axis=-1, keepdims=True)
        hh = (u - m) * lax.rsqrt(v + 1e-5) * lng_ref[...] + lnb_ref[...]
        h_scr[...] = hh
        hsave_ref[...] = hh
        skipo_ref[...] = (jnp.dot(hh, ws_ref[...], preferred_element_type=jnp.float32)
                          + bs_ref[...])

    h = h_scr[...]
    tm_ref[...] = jnp.dot(h, wm_ref[0], preferred_element_type=jnp.float32)
    tgd_ref[...] = jnp.dot(h, wgd_ref[0], preferred_element_type=jnp.float32)
    tgs_ref[...] = jnp.dot(h, wgs_ref[0], preferred_element_type=jnp.float32)


_AGG_SPEC = pl.BlockSpec((2, BN, H), lambda nb, *_: (0, nb, 0))


def _transform_next(skip_prev, agg, ln_g, ln_b, lp):
    nh = jax.ShapeDtypeStruct((N, H), jnp.float32)
    return pl.pallas_call(
        _tf_next_body,
        grid=(NB, NT),
        in_specs=[
            _NODE_SPEC, _AGG_SPEC,
            pl.BlockSpec((1, H), lambda nb, t: (0, 0)),
            pl.BlockSpec((1, H), lambda nb, t: (0, 0)),
        ] + _W_SPECS,
        out_specs=_TAB_SPECS + [_NODE_SPEC, _NODE_SPEC],
        out_shape=_TAB_SHAPES + [nh, nh],
        scratch_shapes=[pltpu.VMEM((BN, H), jnp.float32)],
    )(skip_prev, agg, ln_g.reshape(1, H), ln_b.reshape(1, H),
      lp['W_msg'], lp['W_gd'], lp['W_gs'], lp['W_skip'],
      lp['b_skip'].reshape(1, H))


def _jk_body(skip_ref, agg_ref, h1_ref, h2_ref, jkw_ref, jkb_ref,
             p1a_ref, p1b_ref, a_ref, b_ref):
    u = skip_ref[...] + agg_ref[0] + agg_ref[1]
    hf = (jnp.dot(h1_ref[...], jkw_ref[0], preferred_element_type=jnp.float32)
          + jnp.dot(h2_ref[...], jkw_ref[1], preferred_element_type=jnp.float32)
          + jnp.dot(u, jkw_ref[2], preferred_element_type=jnp.float32)
          + jkb_ref[...])
    a_ref[...] = jnp.dot(hf, p1a_ref[...], preferred_element_type=jnp.float32)
    b_ref[...] = jnp.dot(hf, p1b_ref[...], preferred_element_type=jnp.float32)


def _jk_project(skip2, agg, h1, h2, jk_W, jk_b, p1a, p1b):
    nh = jax.ShapeDtypeStruct((N, H), jnp.float32)
    blk = pl.BlockSpec((BN, H), lambda nb: (nb, 0))
    return pl.pallas_call(
        _jk_body,
        grid=(NB,),
        in_specs=[
            blk, _AGG_SPEC, blk, blk,
            pl.BlockSpec((3, H, H), lambda nb: (0, 0, 0)),
            pl.BlockSpec((1, H), lambda nb: (0, 0)),
            pl.BlockSpec((H, H), lambda nb: (0, 0)),
            pl.BlockSpec((H, H), lambda nb: (0, 0)),
        ],
        out_specs=[blk, blk],
        out_shape=[nh, nh],
    )(skip2, agg, h1, h2, jk_W.reshape(3, H, H), jk_b.reshape(1, H), p1a, p1b)


def _final_body(g_ref, pit_ref, ons_ref, wp_ref, wo_ref, b1_ref,
                w2_ref, b2_ref, w3_ref, b3_ref, out_ref):
    c = pit_ref[...] * wp_ref[...]
    c = c + ons_ref[:, 0:1] * wo_ref[0:1, :] + ons_ref[:, 1:2] * wo_ref[1:2, :]
    z1 = jnp.maximum(g_ref[...] + c + b1_ref[...], 0.0)
    z2 = jnp.maximum(jnp.dot(z1, w2_ref[...], preferred_element_type=jnp.float32)
                     + b2_ref[...], 0.0)
    o = jnp.sum(z2 * w3_ref[...], axis=1, keepdims=True) + b3_ref[...]
    out_ref[...] = 1.0 / (1.0 + jnp.exp(-o))


def _final_mlp(g, pitch, onset, wp, wo, b1, w2, b2, w3, b3):
    return pl.pallas_call(
        _final_body,
        grid=(T // BN,),
        in_specs=[
            pl.BlockSpec((BN, H), lambda i: (i, 0)),
            pl.BlockSpec((BN, 1), lambda i: (i, 0)),
            pl.BlockSpec((BN, 2), lambda i: (i, 0)),
            pl.BlockSpec((1, H), lambda i: (0, 0)),
            pl.BlockSpec((2, H), lambda i: (0, 0)),
            pl.BlockSpec((1, H), lambda i: (0, 0)),
            pl.BlockSpec((H, H // 2), lambda i: (0, 0)),
            pl.BlockSpec((1, H // 2), lambda i: (0, 0)),
            pl.BlockSpec((1, H // 2), lambda i: (0, 0)),
            pl.BlockSpec((1, 1), lambda i: (0, 0)),
        ],
        out_specs=pl.BlockSpec((BN, 1), lambda i: (i, 0)),
        out_shape=jax.ShapeDtypeStruct((T, 1), jnp.float32),
    )(g, pitch, onset, wp, wo, b1, w2, b2, w3, b3)


@functools.partial(
    pl.kernel,
    out_type=jax.ShapeDtypeStruct((2, N_PAD, H), jnp.float32),
    mesh=_sc_mesh,
    scratch_types=[
        pltpu.VMEM((CHUNK,), jnp.int32),
        pltpu.VMEM((CHUNK,), jnp.int32),
        pltpu.VMEM((CHUNK,), jnp.int32),
        pltpu.VMEM((CHUNK, H), jnp.float32),
        pltpu.VMEM((CHUNK, H), jnp.float32),
        pltpu.VMEM((CHUNK, H), jnp.float32),
        pltpu.VMEM((CHUNK, H), jnp.float32),
        pltpu.VMEM_SHARED((N_PAD, H), jnp.float32),
        pltpu.SemaphoreType.DMA,
        pltpu.SemaphoreType.DMA,
        pltpu.SemaphoreType.DMA,
    ],
)
def _edge_kernel(tm_hbm, tgd_hbm, tgs_hbm, isrc_hbm, idst_hbm, dnode_hbm,
                 agg_hbm, isrc_v, idst_v, dnode_v, rm_v, rgd_v, rgs_v, msg_v,
                 acc_sh, sem_m, sem_gd, sem_gs):
    cid = lax.axis_index("c")
    sid = lax.axis_index("s")
    wid = sid * 2 + cid
    row0 = sid * ROWS_PER_TILE

    zero16 = jnp.zeros((16,), jnp.float32)

    def _zrow(r, carry):
        for v in range(H // 16):
            msg_v[r, pl.ds(v * 16, 16)] = zero16
        return carry

    lax.fori_loop(0, CHUNK, _zrow, 0)
    for k in range(ROWS_PER_TILE // CHUNK):
        pltpu.sync_copy(msg_v, acc_sh.at[pl.ds(row0 + k * CHUNK, CHUNK)])
    plsc.subcore_barrier()

    def _chunk(i, carry):
        off = wid * (NCH_E * CHUNK) + i * CHUNK
        pltpu.sync_copy(isrc_hbm.at[pl.ds(off, CHUNK)], isrc_v)
        pltpu.sync_copy(idst_hbm.at[pl.ds(off, CHUNK)], idst_v)
        pltpu.sync_copy(dnode_hbm.at[pl.ds(off, CHUNK)], dnode_v)
        cm = pltpu.async_copy(tm_hbm.at[isrc_v], rm_v, sem_m)
        cgd = pltpu.async_copy(tgd_hbm.at[idst_v], rgd_v, sem_gd)
        cgs = pltpu.async_copy(tgs_hbm.at[isrc_v], rgs_v, sem_gs)
        cm.wait()
        cgd.wait()
        cgs.wait()

        def _row(r, cy):
            for v in range(H // 16):
                sl = pl.ds(v * 16, 16)
                pre = rgd_v[r, sl] + rgs_v[r, sl]
                gate = 1.0 / (1.0 + jnp.exp(-pre))
                msg_v[r, sl] = gate * rm_v[r, sl]
            return cy

        lax.fori_loop(0, CHUNK, _row, 0)
        pltpu.sync_copy(msg_v, acc_sh.at[dnode_v], add=True)
        return carry

    lax.fori_loop(0, NCH_E, _chunk, 0)
    plsc.subcore_barrier()
    pltpu.sync_copy(acc_sh.at[pl.ds(row0, ROWS_PER_TILE)],
                    agg_hbm.at[cid, pl.ds(row0, ROWS_PER_TILE)])


@functools.partial(
    pl.kernel,
    out_type=jax.ShapeDtypeStruct((T_PAD, H), jnp.float32),
    mesh=_sc_mesh,
    scratch_types=[
        pltpu.VMEM((NCH_P * PCHUNK,), jnp.int32),
        pltpu.VMEM((NCH_P * PCHUNK,), jnp.int32),
        pltpu.VMEM((PCHUNK, H), jnp.float32),
        pltpu.VMEM((PCHUNK, H), jnp.float32),
        pltpu.VMEM((PCHUNK, H), jnp.float32),
        pltpu.VMEM((PCHUNK, H), jnp.float32),
        pltpu.SemaphoreType.DMA,
        pltpu.SemaphoreType.DMA,
        pltpu.SemaphoreType.DMA,
        pltpu.SemaphoreType.DMA,
    ],
)
def _pred_gather_kernel(a_hbm, b_hbm, si_hbm, di_hbm, gout_hbm,
                        siloc, diloc, ga0, ga1, gb0, gb1,
                        sa0, sa1, sb0, sb1):
    cid = lax.axis_index("c")
    sid = lax.axis_index("s")
    wid = sid * 2 + cid
    ga = (ga0, ga1)
    gb = (gb0, gb1)
    sa = (sa0, sa1)
    sb = (sb0, sb1)

    npt = NCH_P * PCHUNK
    pltpu.sync_copy(si_hbm.at[pl.ds(wid * npt, npt)], siloc)
    pltpu.sync_copy(di_hbm.at[pl.ds(wid * npt, npt)], diloc)

    descs = {}

    def _issue(i):
        b = i % 2
        descs[(i, 'a')] = pltpu.async_copy(
            a_hbm.at[siloc.at[pl.ds(i * PCHUNK, PCHUNK)]], ga[b], sa[b])
        descs[(i, 'b')] = pltpu.async_copy(
            b_hbm.at[diloc.at[pl.ds(i * PCHUNK, PCHUNK)]], gb[b], sb[b])

    _issue(0)
    base = wid * NCH_P * PCHUNK
    for i in range(NCH_P):
        b = i % 2
        if i < NCH_P - 1:
            _issue(i + 1)
        descs[(i, 'a')].wait()
        descs[(i, 'b')].wait()

        def _row(r, cy):
            for v in range(H // 16):
                sl = pl.ds(v * 16, 16)
                ga[b][r, sl] = ga[b][r, sl] + gb[b][r, sl]
            return cy

        lax.fori_loop(0, PCHUNK, _row, 0)
        pltpu.sync_copy(ga[b], gout_hbm.at[pl.ds(base + i * PCHUNK, PCHUNK)])


def kernel(target_edge_index, x, embed_edge_index, edge_type, pitch_score,
           onset_score, params):
    src = embed_edge_index[0].astype(jnp.int32)
    dst = embed_edge_index[1].astype(jnp.int32)
    et = edge_type.astype(jnp.int32)

    isrc = et * N + src
    idst = et * N + dst

    epad = E_PAD - E
    zpad = jnp.zeros((epad,), jnp.int32)
    isrc_p = jnp.concatenate([isrc, zpad])
    idst_p = jnp.concatenate([idst, zpad])
    dnode_p = jnp.concatenate([dst, jnp.full((epad,), N, jnp.int32)])

    tpad = T_PAD - T
    tz = jnp.zeros((tpad,), jnp.int32)
    si_p = jnp.concatenate([target_edge_index[0].astype(jnp.int32), tz])
    di_p = jnp.concatenate([target_edge_index[1].astype(jnp.int32), tz])

    layers = params['layers']
    ln_g, ln_b = params['ln_g'], params['ln_b']

    tm, tgd, tgs, skip = _transform_first(x, layers[0])
    agg = _edge_kernel(tm, tgd, tgs, isrc_p, idst_p, dnode_p)

    tm, tgd, tgs, skip, h1 = _transform_next(skip, agg, ln_g, ln_b, layers[1])
    agg = _edge_kernel(tm, tgd, tgs, isrc_p, idst_p, dnode_p)

    tm, tgd, tgs, skip, h2 = _transform_next(skip, agg, ln_g, ln_b, layers[2])
    agg = _edge_kernel(tm, tgd, tgs, isrc_p, idst_p, dnode_p)

    p1_W = params['p1_W']
    a_tab, b_tab = _jk_project(skip, agg, h1, h2, params['jk_W'],
                               params['jk_b'], p1_W[:H], p1_W[H:2 * H])

    g = _pred_gather_kernel(a_tab, b_tab, si_p, di_p)

    return _final_mlp(
        g, pitch_score, onset_score,
        p1_W[2 * H:2 * H + 1], p1_W[2 * H + 1:2 * H + 3],
        params['p1_b'].reshape(1, H),
        params['p2_W'], params['p2_b'].reshape(1, H // 2),
        params['p3_W'].reshape(1, H // 2), params['p3_b'].reshape(1, 1))
